# baseline (speedup 1.0000x reference)
import jax
import jax.numpy as jnp
from jax.experimental import pallas as pl
from jax.experimental.pallas import tpu as pltpu

B, S = 16, 512
IN_DIM, ATTN_DIM, DEPTH, HEADS, DH = 512, 512, 4, 8, 64
CB_DIM, CB_SIZE = 64, 8192
FF_DIM = 4 * ATTN_DIM
CB_CHUNK = 2048
_HI = jax.lax.Precision.HIGHEST
_F32 = jnp.float32
_BF16 = jnp.bfloat16


def _layernorm(t, g, b):
    mu = jnp.mean(t, axis=-1, keepdims=True)
    v = jnp.mean((t - mu) ** 2, axis=-1, keepdims=True)
    return (t - mu) / jnp.sqrt(v + 1e-5) * g + b


def _mixed_dot(lhs_f32, rhs_bf16):
    hi = lhs_f32.astype(_BF16)
    lo = (lhs_f32 - hi.astype(_F32)).astype(_BF16)
    return (jnp.dot(hi, rhs_bf16, preferred_element_type=_F32)
            + jnp.dot(lo, rhs_bf16, preferred_element_type=_F32))


def _f32_dot(lhs_f32, rhs_f32):
    return jnp.dot(lhs_f32, rhs_f32, preferred_element_type=_F32,
                   precision=_HI)


def _encoder_body(x_ref, w_in_ref, posb_ref, lnstack_ref, ffb1_ref,
                  wq_ref, wk_ref, wv_ref, wo_ref, ffw1_ref, ffw2_ref,
                  lnf_ref, wout_ref, bout_ref, z_ref, h_ref):
    d = pl.program_id(0)
    b = pl.program_id(1)

    @pl.when(d == 0)
    def _():
        h_ref[b] = _f32_dot(x_ref[0], w_in_ref[...]) + posb_ref[...]

    h = h_ref[b]

    g1 = lnstack_ref[0, 0:1, :]
    b1 = lnstack_ref[0, 1:2, :]
    g2 = lnstack_ref[0, 2:3, :]
    b2 = lnstack_ref[0, 3:4, :]
    ffb2 = lnstack_ref[0, 4:5, :]

    ln1 = _layernorm(h, g1, b1)
    q = _mixed_dot(ln1, wq_ref[0]).astype(_BF16)
    k = _mixed_dot(ln1, wk_ref[0]).astype(_BF16)
    v = _mixed_dot(ln1, wv_ref[0]).astype(_BF16)

    scale = DH ** -0.5
    o_parts = []
    for hh in range(HEADS):
        sl = slice(hh * DH, (hh + 1) * DH)
        s = jax.lax.dot_general(
            q[:, sl], k[:, sl], (((1,), (1,)), ((), ())),
            preferred_element_type=_F32) * scale
        m = jnp.max(s, axis=-1, keepdims=True)
        e = jnp.exp(s - m)
        a = (e / jnp.sum(e, axis=-1, keepdims=True)).astype(_BF16)
        o_parts.append(
            jnp.dot(a, v[:, sl], preferred_element_type=_F32).astype(_BF16))
    o = jnp.concatenate(o_parts, axis=1)

    h = h + jnp.dot(o, wo_ref[0], preferred_element_type=_F32)

    ln2 = _layernorm(h, g2, b2)
    gf = _mixed_dot(ln2, ffw1_ref[0]) + ffb1_ref[0, 0:1, :]
    gb = jax.nn.gelu(gf).astype(_BF16)
    h = h + jnp.dot(gb, ffw2_ref[0], preferred_element_type=_F32) + ffb2

    h_ref[b] = h

    @pl.when(d == DEPTH - 1)
    def _():
        hf = _layernorm(h, lnf_ref[0:1, :], lnf_ref[1:2, :])
        z_ref[0] = _f32_dot(hf, wout_ref[...]) + bout_ref[...]


def _encoder_call(x, w_in, posb, lnstack, ffb1, wq, wk, wv, wo,
                  ffw1, ffw2, lnf, w_out, b_out, interpret=False):
    return pl.pallas_call(
        _encoder_body,
        out_shape=jax.ShapeDtypeStruct((B, S, CB_DIM), jnp.float32),
        grid=(DEPTH, B),
        in_specs=[
            pl.BlockSpec((1, S, IN_DIM), lambda d, b: (b, 0, 0)),
            pl.BlockSpec((IN_DIM, ATTN_DIM), lambda d, b: (0, 0)),
            pl.BlockSpec((S, ATTN_DIM), lambda d, b: (0, 0)),
            pl.BlockSpec((1, 5, ATTN_DIM), lambda d, b: (d, 0, 0)),
            pl.BlockSpec((1, 1, FF_DIM), lambda d, b: (d, 0, 0)),
            pl.BlockSpec((1, ATTN_DIM, ATTN_DIM), lambda d, b: (d, 0, 0)),
            pl.BlockSpec((1, ATTN_DIM, ATTN_DIM), lambda d, b: (d, 0, 0)),
            pl.BlockSpec((1, ATTN_DIM, ATTN_DIM), lambda d, b: (d, 0, 0)),
            pl.BlockSpec((1, ATTN_DIM, ATTN_DIM), lambda d, b: (d, 0, 0)),
            pl.BlockSpec((1, ATTN_DIM, FF_DIM), lambda d, b: (d, 0, 0)),
            pl.BlockSpec((1, FF_DIM, ATTN_DIM), lambda d, b: (d, 0, 0)),
            pl.BlockSpec((2, ATTN_DIM), lambda d, b: (0, 0)),
            pl.BlockSpec((ATTN_DIM, CB_DIM), lambda d, b: (0, 0)),
            pl.BlockSpec((1, CB_DIM), lambda d, b: (0, 0)),
        ],
        out_specs=pl.BlockSpec((1, S, CB_DIM), lambda d, b: (b, 0, 0)),
        scratch_shapes=[pltpu.VMEM((B, S, ATTN_DIM), jnp.float32)],
        compiler_params=pltpu.CompilerParams(
            dimension_semantics=("arbitrary", "parallel"),
            vmem_limit_bytes=60 * 1024 * 1024,
        ),
        name="extractor_encoder",
        interpret=interpret,
    )(x, w_in, posb, lnstack, ffb1, wq, wk, wv, wo, ffw1, ffw2, lnf,
      w_out, b_out)


def _vq_body(z_ref, cbt_ref, cb_ref, qsum_ref, closs_ref):
    z = z_ref[0]
    cbsq = jnp.sum(cbt_ref[...] * cbt_ref[...], axis=0, keepdims=True)

    best = None
    best_idx = None
    for c in range(CB_SIZE // CB_CHUNK):
        sl = slice(c * CB_CHUNK, (c + 1) * CB_CHUNK)
        sc = _f32_dot(z, cbt_ref[:, sl])
        d2 = cbsq[:, sl] - 2.0 * sc
        mc = jnp.min(d2, axis=-1, keepdims=True)
        iota = jax.lax.broadcasted_iota(jnp.int32, (S, CB_CHUNK), 1) + c * CB_CHUNK
        idxc = jnp.min(jnp.where(d2 == mc, iota, CB_SIZE), axis=-1, keepdims=True)
        if best is None:
            best, best_idx = mc, idxc
        else:
            better = mc < best
            best = jnp.where(better, mc, best)
            best_idx = jnp.where(better, idxc, best_idx)

    q = jnp.zeros((S, CB_DIM), jnp.float32)
    for c in range(CB_SIZE // CB_CHUNK):
        iota = jax.lax.broadcasted_iota(jnp.int32, (S, CB_CHUNK), 1) + c * CB_CHUNK
        oh = (iota == best_idx).astype(jnp.float32)
        q = q + _f32_dot(oh, cb_ref[c * CB_CHUNK:(c + 1) * CB_CHUNK, :])

    qsum_ref[0] = jnp.sum(q, axis=0, keepdims=True)
    diff = q - z
    closs_ref[0] = jnp.broadcast_to(jnp.sum(diff * diff), (1, CB_DIM))


def _vq_call(z, cbt, cb, interpret=False):
    return pl.pallas_call(
        _vq_body,
        out_shape=(
            jax.ShapeDtypeStruct((B, 1, CB_DIM), jnp.float32),
            jax.ShapeDtypeStruct((B, 1, CB_DIM), jnp.float32),
        ),
        grid=(B,),
        in_specs=[
            pl.BlockSpec((1, S, CB_DIM), lambda b: (b, 0, 0)),
            pl.BlockSpec((CB_DIM, CB_SIZE), lambda b: (0, 0)),
            pl.BlockSpec((CB_SIZE, CB_DIM), lambda b: (0, 0)),
        ],
        out_specs=(
            pl.BlockSpec((1, 1, CB_DIM), lambda b: (b, 0, 0)),
            pl.BlockSpec((1, 1, CB_DIM), lambda b: (b, 0, 0)),
        ),
        compiler_params=pltpu.CompilerParams(
            dimension_semantics=("parallel",),
            vmem_limit_bytes=60 * 1024 * 1024,
        ),
        name="extractor_vq",
        interpret=interpret,
    )(z, cbt, cb)


def kernel(x, w_in, b_in, pos_emb, ln1_g, ln1_b, wq, wk, wv, wo,
           ln2_g, ln2_b, ff_w1, ff_b1, ff_w2, ff_b2, lnf_g, lnf_b,
           w_out, b_out, codebook, interpret=False):
    posb = pos_emb[:S] + b_in[None, :]
    lnstack = jnp.stack([ln1_g, ln1_b, ln2_g, ln2_b, ff_b2], axis=1)
    ffb1 = ff_b1[:, None, :]
    lnf = jnp.stack([lnf_g, lnf_b], axis=0)
    z = _encoder_call(x, w_in, posb, lnstack, ffb1,
                      wq.astype(_BF16), wk.astype(_BF16),
                      wv.astype(_BF16), wo.astype(_BF16),
                      ff_w1.astype(_BF16), ff_w2.astype(_BF16),
                      lnf, w_out, b_out[None, :],
                      interpret=interpret)
    qsum, closs = _vq_call(z, codebook.T, codebook, interpret=interpret)
    out = qsum[:, 0, :]
    commit_loss = jnp.sum(closs[:, 0, 0]) / (B * S * CB_DIM)
    return out, commit_loss
